# baseline (speedup 1.0000x reference)
import jax
import jax.numpy as jnp
from jax import lax
from jax.experimental import pallas as pl
from jax.experimental.pallas import tpu as pltpu
from jax.experimental.pallas import tpu_sc as plsc

N_TOK = 10
D = 64
DP = 128
B = 1024
S = 200
V = 1000000
SEQ_G = S - N_TOK
HALF = SEQ_G // 2
G = 96

NC = 2
NS = 16
NW = NC * NS
BPW = B // NW
CH = 2
NCH = BPW // CH

def _soft_embed_sc(table, idx95, learned, out,
                   idx_v0, idx_v1, buf0, buf1, sem0, sem1):
    wid = lax.axis_index("s") * NC + lax.axis_index("c")
    idxs = (idx_v0, idx_v1)
    bufs = (buf0, buf1)
    sems = (sem0, sem1)

    for nb in range(2):
        for j in range(CH):
            pltpu.sync_copy(learned, bufs[nb].at[pl.ds(j * S, N_TOK), pl.ds(0, D)])

    def fetch(c, nb):
        b0 = wid * BPW + c * CH
        pltpu.sync_copy(idx95.at[pl.ds(b0 * 2, CH * 2), pl.ds(0, G)],
                        idxs[nb])
        dmas = []
        for j in range(CH * 2):
            row0 = (j // 2) * S + (N_TOK if j % 2 == 0 else S - G)
            dst = bufs[nb].at[pl.ds(row0, G)]
            src = table.at[idxs[nb].at[j]]
            dmas.append(pltpu.async_copy(src, dst, sems[nb]))
        return dmas

    pending = fetch(0, 0)
    for c in range(NCH):
        nb = c % 2
        nxt = fetch(c + 1, 1 - nb) if c + 1 < NCH else None
        for d in pending:
            d.wait()
        b0 = wid * BPW + c * CH
        for j in range(CH):
            pltpu.sync_copy(bufs[nb].at[pl.ds(j * S, S), pl.ds(0, D)],
                            out.at[b0 + j])
        pending = nxt


def kernel(tokens, wte_weight, learned_embedding):
    idxc = jnp.minimum(tokens, V - 1)
    idx96 = jnp.pad(
        jnp.stack([idxc[:, N_TOK:N_TOK + G], idxc[:, S - G:]], 1)
        .reshape(B * 2, G),
        ((0, 0), (0, DP - G)))
    mesh = plsc.VectorSubcoreMesh(core_axis_name="c", subcore_axis_name="s")

    eye_pad = jnp.eye(D, DP, dtype=jnp.float32)
    table_pad = jnp.dot(wte_weight, eye_pad)

    emb = pl.kernel(
        _soft_embed_sc,
        mesh=mesh,
        compiler_params=pltpu.CompilerParams(use_tc_tiling_on_sc=False),
        out_type=jax.ShapeDtypeStruct((B, S, D), jnp.float32),
        scratch_types=[
            pltpu.VMEM((CH * 2, G), jnp.int32),
            pltpu.VMEM((CH * 2, G), jnp.int32),
            pltpu.VMEM((CH * S, DP), jnp.float32),
            pltpu.VMEM((CH * S, DP), jnp.float32),
            pltpu.SemaphoreType.DMA,
            pltpu.SemaphoreType.DMA,
        ],
    )
    return emb(table_pad, idx96, learned_embedding)
